# baseline (speedup 1.0000x reference)
import functools

import jax
import jax.numpy as jnp
from jax import lax
from jax.experimental import pallas as pl
from jax.experimental.pallas import tpu as pltpu
from jax.experimental.pallas import tpu_sc as plsc

N = 10000
E = 320000
P = 10240
NC, NS = 2, 16
NW = NC * NS
K = 128
ER = E // K
RPT = P // NS
R16 = 78
R128 = 156

_MESH = plsc.VectorSubcoreMesh(core_axis_name="c", subcore_axis_name="s",
                               num_cores=NC, num_subcores=NS)


def _zero_rows_f32(ref, nrows, ncols):
    z = jnp.zeros((16,), jnp.float32)

    @pl.loop(0, ncols // 16)
    def _(j):
        @pl.loop(0, nrows)
        def _(r):
            ref[r, pl.ds(j * 16, 16)] = z


def _zero_rows_bf16(ref, nrows, ncols):
    z = jnp.zeros((32,), jnp.bfloat16)

    @pl.loop(0, ncols // 32)
    def _(j):
        @pl.loop(0, nrows)
        def _(r):
            ref[r, pl.ds(j * 32, 32)] = z


def _ring(issue_gather, wait_table, srcall, dstall, rows, acc,
          gsems, ssems, nb):
    L = 4
    for j in range(L - 1):
        issue_gather(j, rows[j], gsems[j])

    @pl.loop(0, nb, step=L)
    def _(base):
        for u in range(L):
            b = base + u
            j = u
            pltpu.make_async_copy(wait_table.at[srcall.at[b]], rows[j],
                                  gsems[j]).wait()
            pltpu.async_copy(rows[j], acc.at[dstall.at[b]], ssems[j],
                             add=True)
            jp = (j + L - 1) % L

            @pl.when(b >= 1)
            def _():
                pltpu.make_async_copy(rows[jp], acc.at[dstall.at[b - 1]],
                                      ssems[jp]).wait()

            @pl.when(b + L - 1 < nb)
            def _():
                issue_gather(b + L - 1, rows[jp], gsems[jp])

    pltpu.make_async_copy(rows[(nb - 1) % L], acc.at[dstall.at[nb - 1]],
                          ssems[(nb - 1) % L]).wait()


def _sc_deg_body(dst_hbm, out_hbm, dflat, hist, rbuf, pbuf, shared, sem):
    c = lax.axis_index("c")
    s = lax.axis_index("s")
    wid = c * NS + s
    nper = R16 * K

    pltpu.sync_copy(dst_hbm.at[pl.ds(wid * nper, nper)], dflat)

    z = jnp.zeros((16,), jnp.float32)

    @pl.loop(0, P // 16)
    def _(m):
        hist[pl.ds(m * 16, 16)] = z

    ones = jnp.ones((16,), jnp.float32)

    @pl.loop(0, nper // 16)
    def _(j):
        plsc.addupdate_scatter(hist, [dflat[pl.ds(j * 16, 16)]], ones)

    @pl.when(wid < 4)
    def _():
        pltpu.sync_copy(dst_hbm.at[pl.ds(NW * nper + wid * K, K)],
                        dflat.at[pl.ds(0, K)])

        @pl.loop(0, K // 16)
        def _(j):
            plsc.addupdate_scatter(hist, [dflat[pl.ds(j * 16, 16)]], ones)

    pltpu.sync_copy(hist, shared.at[pl.ds(s * P, P)])
    plsc.subcore_barrier()
    for t in range(NS):
        pltpu.async_copy(shared.at[pl.ds(t * P + s * RPT, RPT)],
                         rbuf.at[pl.ds(t * RPT, RPT)], sem)
    for t in range(NS):
        pltpu.make_async_copy(shared.at[pl.ds(t * P + s * RPT, RPT)],
                              rbuf.at[pl.ds(t * RPT, RPT)], sem).wait()
    for g in range(RPT // 16):
        d = jnp.zeros((16,), jnp.float32)
        for r in range(NS):
            d = d + rbuf[pl.ds(r * RPT + g * 16, 16)]
        pbuf[pl.ds(16 * g, 16)] = d
    pltpu.sync_copy(pbuf, out_hbm.at[pl.ds(wid * RPT, RPT)])


_deg_call = functools.partial(
    pl.kernel,
    out_type=jax.ShapeDtypeStruct((2 * P,), jnp.float32),
    mesh=_MESH,
    compiler_params=pltpu.CompilerParams(use_tc_tiling_on_sc=False,
                                         needs_layout_passes=False),
    scratch_types=[
        pltpu.VMEM((R16 * K,), jnp.int32),
        pltpu.VMEM((P,), jnp.float32),
        pltpu.VMEM((NS * RPT,), jnp.float32),
        pltpu.VMEM((RPT,), jnp.float32),
        pltpu.VMEM_SHARED((NS * P,), jnp.float32),
        pltpu.SemaphoreType.DMA,
    ],
)(_sc_deg_body)


def _sc_agg128_body(hlo_hbm, hhi_hbm, e3_hbm, out_hbm,
                    srcall, dstall, xsrc, xdst,
                    rows0, rows1, rows2, rows3, acc,
                    gs0, gs1, gs2, gs3, ss0, ss1, ss2, ss3):
    c = lax.axis_index("c")
    s = lax.axis_index("s")

    pltpu.sync_copy(e3_hbm.at[0, pl.ds(s * R128, R128)], srcall)
    pltpu.sync_copy(e3_hbm.at[1, pl.ds(s * R128, R128)], dstall)

    _zero_rows_bf16(rows0, K, 64)
    for t in range(RPT // K):
        pltpu.sync_copy(rows0, acc.at[pl.ds(s * RPT + t * K, K)])
    plsc.subcore_barrier()

    def issue(b, buf, sem):
        @pl.when(c == 0)
        def _():
            pltpu.async_copy(hlo_hbm.at[srcall.at[b]], buf, sem)

        @pl.when(c == 1)
        def _():
            pltpu.async_copy(hhi_hbm.at[srcall.at[b]], buf, sem)

    _ring(issue, hlo_hbm, srcall, dstall, [rows0, rows1, rows2, rows3], acc,
          [gs0, gs1, gs2, gs3], [ss0, ss1, ss2, ss3], R128)

    @pl.when(s < 4)
    def _():
        pltpu.sync_copy(e3_hbm.at[0, ER - 4 + s], xsrc)
        pltpu.sync_copy(e3_hbm.at[1, ER - 4 + s], xdst)

        @pl.when(c == 0)
        def _():
            pltpu.async_copy(hlo_hbm.at[xsrc], rows0, gs0).wait()

        @pl.when(c == 1)
        def _():
            pltpu.async_copy(hhi_hbm.at[xsrc], rows0, gs0).wait()

        pltpu.sync_copy(rows0, acc.at[xdst], add=True)

    plsc.subcore_barrier()
    pltpu.sync_copy(acc.at[pl.ds(s * RPT, RPT)],
                    out_hbm.at[pl.ds(c * P + s * RPT, RPT)])


_agg128_call = functools.partial(
    pl.kernel,
    out_type=jax.ShapeDtypeStruct((2 * P, 64), jnp.bfloat16),
    mesh=_MESH,
    compiler_params=pltpu.CompilerParams(use_tc_tiling_on_sc=False),
    scratch_types=[
        pltpu.VMEM((R128, K), jnp.int32),
        pltpu.VMEM((R128, K), jnp.int32),
        pltpu.VMEM((K,), jnp.int32),
        pltpu.VMEM((K,), jnp.int32),
        pltpu.VMEM((K, 64), jnp.bfloat16),
        pltpu.VMEM((K, 64), jnp.bfloat16),
        pltpu.VMEM((K, 64), jnp.bfloat16),
        pltpu.VMEM((K, 64), jnp.bfloat16),
        pltpu.VMEM_SHARED((P, 64), jnp.bfloat16),
    ] + [pltpu.SemaphoreType.DMA] * 8,
)(_sc_agg128_body)


def _sc_agg16_body(h_hbm, e3_hbm, out_hbm,
                   srcall, dstall, xsrc, xdst,
                   rows0, rows1, rows2, rows3, acc,
                   gs0, gs1, gs2, gs3, ss0, ss1, ss2, ss3):
    c = lax.axis_index("c")
    s = lax.axis_index("s")
    wid = c * NS + s

    pltpu.sync_copy(e3_hbm.at[0, pl.ds(wid * R16, R16)], srcall)
    pltpu.sync_copy(e3_hbm.at[1, pl.ds(wid * R16, R16)], dstall)

    _zero_rows_f32(rows0, K, 16)
    for t in range(RPT // K):
        pltpu.sync_copy(rows0, acc.at[pl.ds(s * RPT + t * K, K)])
    plsc.subcore_barrier()

    def issue(b, buf, sem):
        pltpu.async_copy(h_hbm.at[srcall.at[b]], buf, sem)

    RMAIN = 76
    _ring(issue, h_hbm, srcall, dstall, [rows0, rows1, rows2, rows3], acc,
          [gs0, gs1, gs2, gs3], [ss0, ss1, ss2, ss3], RMAIN)

    for b in (76, 77):
        pltpu.async_copy(h_hbm.at[srcall.at[b]], rows0, gs0).wait()
        pltpu.sync_copy(rows0, acc.at[dstall.at[b]], add=True)

    @pl.when(wid < 4)
    def _():
        pltpu.sync_copy(e3_hbm.at[0, ER - 4 + wid], xsrc)
        pltpu.sync_copy(e3_hbm.at[1, ER - 4 + wid], xdst)
        pltpu.async_copy(h_hbm.at[xsrc], rows0, gs0).wait()
        pltpu.sync_copy(rows0, acc.at[xdst], add=True)

    plsc.subcore_barrier()
    pltpu.sync_copy(acc.at[pl.ds(s * RPT, RPT)],
                    out_hbm.at[pl.ds(c * P + s * RPT, RPT)])


_agg16_call = functools.partial(
    pl.kernel,
    out_type=jax.ShapeDtypeStruct((2 * P, 16), jnp.float32),
    mesh=_MESH,
    compiler_params=pltpu.CompilerParams(use_tc_tiling_on_sc=False),
    scratch_types=[
        pltpu.VMEM((R16, K), jnp.int32),
        pltpu.VMEM((R16, K), jnp.int32),
        pltpu.VMEM((K,), jnp.int32),
        pltpu.VMEM((K,), jnp.int32),
        pltpu.VMEM((K, 16), jnp.float32),
        pltpu.VMEM((K, 16), jnp.float32),
        pltpu.VMEM((K, 16), jnp.float32),
        pltpu.VMEM((K, 16), jnp.float32),
        pltpu.VMEM_SHARED((P, 16), jnp.float32),
    ] + [pltpu.SemaphoreType.DMA] * 8,
)(_sc_agg16_body)


BLK = 1024
GRID = P // BLK
BLKO = 2048
GRIDO = P // BLKO


def _pk_to_col(d, nrows):
    g = nrows // K
    x = jnp.broadcast_to(d[:, None, :], (g, K, K)).reshape(nrows, K)
    l = lax.broadcasted_iota(jnp.int32, (nrows, K), 1)
    n = lax.broadcasted_iota(jnp.int32, (nrows, K), 0)
    return jnp.sum(jnp.where(l == n % K, x, 0.0), axis=1, keepdims=True)


def _tc_mm1a_body(x_ref, w1_ref, h_ref):
    i = pl.program_id(0)
    h = jnp.dot(x_ref[...], w1_ref[...], preferred_element_type=jnp.float32)
    row = i * BLK + lax.broadcasted_iota(jnp.int32, (BLK, 128), 0)
    h_ref[...] = jnp.where(row < N, h, 0.0)


def _tc_mm1a(x, W1):
    return pl.pallas_call(
        _tc_mm1a_body,
        grid=(GRID,),
        in_specs=[
            pl.BlockSpec((BLK, 128), lambda i: (i, 0)),
            pl.BlockSpec((128, 128), lambda i: (0, 0)),
        ],
        out_specs=pl.BlockSpec((BLK, 128), lambda i: (i, 0)),
        out_shape=jax.ShapeDtypeStruct((P, 128), jnp.float32),
    )(x, W1)


def _tc_mm1b_body(h_ref, d0_ref, d1_ref, hlo_ref, hhi_ref, dis_ref):
    dis = lax.rsqrt(d0_ref[...] + d1_ref[...] + 1.0)
    dis_ref[...] = dis
    h = h_ref[...] * _pk_to_col(dis, BLK)
    hb = h.astype(jnp.bfloat16)
    hlo_ref[...] = hb[:, :64]
    hhi_ref[...] = hb[:, 64:]


def _tc_mm1b(hraw, deg_pk):
    return pl.pallas_call(
        _tc_mm1b_body,
        grid=(GRID,),
        in_specs=[
            pl.BlockSpec((BLK, 128), lambda i: (i, 0)),
            pl.BlockSpec((BLK // K, K), lambda i: (i, 0)),
            pl.BlockSpec((BLK // K, K), lambda i: (i + GRID, 0)),
        ],
        out_specs=[
            pl.BlockSpec((BLK, 64), lambda i: (i, 0)),
            pl.BlockSpec((BLK, 64), lambda i: (i, 0)),
            pl.BlockSpec((BLK // K, K), lambda i: (i, 0)),
        ],
        out_shape=[
            jax.ShapeDtypeStruct((P, 64), jnp.bfloat16),
            jax.ShapeDtypeStruct((P, 64), jnp.bfloat16),
            jax.ShapeDtypeStruct((P // K, K), jnp.float32),
        ],
    )(hraw, deg_pk, deg_pk)


def _tc_mm2_body(alo_ref, ahi_ref, hlo_ref, hhi_ref, dis_ref, w2_ref, b1_ref,
                 out_ref):
    dis1 = _pk_to_col(dis_ref[...], BLK)
    b1 = b1_ref[...]
    w2 = w2_ref[...]
    pre_lo = ((alo_ref[...].astype(jnp.float32)
               + hlo_ref[...].astype(jnp.float32)) * dis1 + b1[:, :64])
    pre_hi = ((ahi_ref[...].astype(jnp.float32)
               + hhi_ref[...].astype(jnp.float32)) * dis1 + b1[:, 64:])
    o_lo = jnp.maximum(pre_lo, 0.0)
    o_hi = jnp.maximum(pre_hi, 0.0)
    h2 = (jnp.dot(o_lo, w2[:64], preferred_element_type=jnp.float32)
          + jnp.dot(o_hi, w2[64:], preferred_element_type=jnp.float32))
    out_ref[...] = h2 * dis1


def _tc_mm2(a128, hlo, hhi, dis, W2, b1):
    return pl.pallas_call(
        _tc_mm2_body,
        grid=(GRID,),
        in_specs=[
            pl.BlockSpec((BLK, 64), lambda i: (i, 0)),
            pl.BlockSpec((BLK, 64), lambda i: (i + GRID, 0)),
            pl.BlockSpec((BLK, 64), lambda i: (i, 0)),
            pl.BlockSpec((BLK, 64), lambda i: (i, 0)),
            pl.BlockSpec((BLK // K, K), lambda i: (i, 0)),
            pl.BlockSpec((128, 16), lambda i: (0, 0)),
            pl.BlockSpec((1, 128), lambda i: (0, 0)),
        ],
        out_specs=pl.BlockSpec((BLK, 16), lambda i: (i, 0)),
        out_shape=jax.ShapeDtypeStruct((P, 16), jnp.float32),
    )(a128, a128, hlo, hhi, dis, W2, b1)


def _tc_out_body(p0_ref, p1_ref, h2_ref, dis_ref, b2_ref, out_ref):
    dis1 = _pk_to_col(dis_ref[...], BLKO)
    out_ref[...] = (p0_ref[...] + p1_ref[...] + h2_ref[...]) * dis1 + b2_ref[...]


def _tc_out(acc2, h2p, dis, b2):
    return pl.pallas_call(
        _tc_out_body,
        grid=(GRIDO,),
        in_specs=[
            pl.BlockSpec((BLKO, 16), lambda i: (i, 0)),
            pl.BlockSpec((BLKO, 16), lambda i: (i + GRIDO, 0)),
            pl.BlockSpec((BLKO, 16), lambda i: (i, 0)),
            pl.BlockSpec((BLKO // K, K), lambda i: (i, 0)),
            pl.BlockSpec((1, 16), lambda i: (0, 0)),
        ],
        out_specs=pl.BlockSpec((BLKO, 16), lambda i: (i, 0)),
        out_shape=jax.ShapeDtypeStruct((N, 16), jnp.float32),
    )(acc2, acc2, h2p, dis, b2)


@jax.jit
def kernel(x, edge_index, W1, b1, W2, b2):
    e3 = edge_index.reshape(2, ER, K)

    deg_pk = _deg_call(edge_index[1]).reshape(2 * P // K, K)
    hraw = _tc_mm1a(x, W1)
    hlo, hhi, dis = _tc_mm1b(hraw, deg_pk)

    acc1 = _agg128_call(hlo, hhi, e3)
    h2p = _tc_mm2(acc1, hlo, hhi, dis, W2, b1.reshape(1, 128))

    acc2 = _agg16_call(h2p, e3)
    return _tc_out(acc2, h2p, dis, b2.reshape(1, 16))


# speedup vs baseline: 1.0216x; 1.0216x over previous
import functools

import jax
import jax.numpy as jnp
from jax import lax
from jax.experimental import pallas as pl
from jax.experimental.pallas import tpu as pltpu
from jax.experimental.pallas import tpu_sc as plsc

N = 10000
E = 320000
P = 10240
NC, NS = 2, 16
NW = NC * NS
K = 128
ER = E // K
RPT = P // NS
R16 = 78
R128 = 156

_MESH = plsc.VectorSubcoreMesh(core_axis_name="c", subcore_axis_name="s",
                               num_cores=NC, num_subcores=NS)


def _zero_rows_f32(ref, nrows, ncols):
    z = jnp.zeros((16,), jnp.float32)

    @pl.loop(0, ncols // 16)
    def _(j):
        @pl.loop(0, nrows)
        def _(r):
            ref[r, pl.ds(j * 16, 16)] = z


def _zero_rows_bf16(ref, nrows, ncols):
    z = jnp.zeros((32,), jnp.bfloat16)

    @pl.loop(0, ncols // 32)
    def _(j):
        @pl.loop(0, nrows)
        def _(r):
            ref[r, pl.ds(j * 32, 32)] = z


def _ring(issue_gather, wait_table, srcall, dstall, rows, acc,
          gsems, ssems, nb):
    L = 4
    for j in range(L - 1):
        issue_gather(j, rows[j], gsems[j])

    @pl.loop(0, nb, step=L)
    def _(base):
        for u in range(L):
            b = base + u
            j = u
            pltpu.make_async_copy(wait_table.at[srcall.at[b]], rows[j],
                                  gsems[j]).wait()
            pltpu.async_copy(rows[j], acc.at[dstall.at[b]], ssems[j],
                             add=True)
            jp = (j + L - 1) % L

            @pl.when(b >= 1)
            def _():
                pltpu.make_async_copy(rows[jp], acc.at[dstall.at[b - 1]],
                                      ssems[jp]).wait()

            @pl.when(b + L - 1 < nb)
            def _():
                issue_gather(b + L - 1, rows[jp], gsems[jp])

    pltpu.make_async_copy(rows[(nb - 1) % L], acc.at[dstall.at[nb - 1]],
                          ssems[(nb - 1) % L]).wait()


def _sc_deg_body(e3_hbm, out_hbm, dstall, xdst, ones, zbuf, dbuf, pbuf, acc):
    c = lax.axis_index("c")
    s = lax.axis_index("s")
    wid = c * NS + s

    pltpu.sync_copy(e3_hbm.at[1, pl.ds(wid * R16, R16)], dstall)

    @pl.loop(0, K)
    def _(r):
        ones[r, :] = jnp.ones((16,), jnp.float32)

    _zero_rows_f32(zbuf, K, 16)
    for t in range(RPT // K):
        pltpu.sync_copy(zbuf, acc.at[pl.ds(s * RPT + t * K, K)])
    plsc.subcore_barrier()

    @pl.loop(0, R16)
    def _(b):
        pltpu.sync_copy(ones, acc.at[dstall.at[b]], add=True)

    @pl.when(wid < 4)
    def _():
        pltpu.sync_copy(e3_hbm.at[1, ER - 4 + wid], xdst)
        pltpu.sync_copy(ones, acc.at[xdst], add=True)

    plsc.subcore_barrier()
    pltpu.sync_copy(acc.at[pl.ds(s * RPT, RPT)], dbuf)
    lane = lax.iota(jnp.int32, 16)
    for g in range(RPT // 16):
        d = jnp.zeros((16,), jnp.float32)
        for r in range(16):
            d = jnp.where(lane == r, dbuf[16 * g + r, :], d)
        pbuf[g // 8, pl.ds(16 * (g % 8), 16)] = d
    pltpu.sync_copy(
        pbuf, out_hbm.at[pl.ds((c * NS + s) * (RPT // K), RPT // K)])


_deg_call = functools.partial(
    pl.kernel,
    out_type=jax.ShapeDtypeStruct((2 * P // K, K), jnp.float32),
    mesh=_MESH,
    compiler_params=pltpu.CompilerParams(use_tc_tiling_on_sc=False),
    scratch_types=[
        pltpu.VMEM((R16, K), jnp.int32),
        pltpu.VMEM((K,), jnp.int32),
        pltpu.VMEM((K, 16), jnp.float32),
        pltpu.VMEM((K, 16), jnp.float32),
        pltpu.VMEM((RPT, 16), jnp.float32),
        pltpu.VMEM((RPT // K, K), jnp.float32),
        pltpu.VMEM_SHARED((P, 16), jnp.float32),
    ],
)(_sc_deg_body)


def _sc_agg128_body(hlo_hbm, hhi_hbm, e3_hbm, out_hbm,
                    srcall, dstall, xsrc, xdst,
                    rows0, rows1, rows2, rows3, acc, htab,
                    gs0, gs1, gs2, gs3, ss0, ss1, ss2, ss3):
    c = lax.axis_index("c")
    s = lax.axis_index("s")

    pltpu.sync_copy(e3_hbm.at[0, pl.ds(s * R128, R128)], srcall)
    pltpu.sync_copy(e3_hbm.at[1, pl.ds(s * R128, R128)], dstall)

    @pl.when(c == 0)
    def _():
        pltpu.sync_copy(hlo_hbm.at[pl.ds(s * RPT, RPT)],
                        htab.at[pl.ds(s * RPT, RPT)])

    @pl.when(c == 1)
    def _():
        pltpu.sync_copy(hhi_hbm.at[pl.ds(s * RPT, RPT)],
                        htab.at[pl.ds(s * RPT, RPT)])

    _zero_rows_bf16(rows0, K, 64)
    for t in range(RPT // K):
        pltpu.sync_copy(rows0, acc.at[pl.ds(s * RPT + t * K, K)])
    plsc.subcore_barrier()

    def issue(b, buf, sem):
        pltpu.async_copy(htab.at[srcall.at[b]], buf, sem)

    _ring(issue, htab, srcall, dstall, [rows0, rows1, rows2, rows3], acc,
          [gs0, gs1, gs2, gs3], [ss0, ss1, ss2, ss3], R128)

    @pl.when(s < 4)
    def _():
        pltpu.sync_copy(e3_hbm.at[0, ER - 4 + s], xsrc)
        pltpu.sync_copy(e3_hbm.at[1, ER - 4 + s], xdst)
        pltpu.async_copy(htab.at[xsrc], rows0, gs0).wait()
        pltpu.sync_copy(rows0, acc.at[xdst], add=True)

    plsc.subcore_barrier()
    pltpu.sync_copy(acc.at[pl.ds(s * RPT, RPT)],
                    out_hbm.at[pl.ds(c * P + s * RPT, RPT)])


_agg128_call = functools.partial(
    pl.kernel,
    out_type=jax.ShapeDtypeStruct((2 * P, 64), jnp.bfloat16),
    mesh=_MESH,
    compiler_params=pltpu.CompilerParams(use_tc_tiling_on_sc=False),
    scratch_types=[
        pltpu.VMEM((R128, K), jnp.int32),
        pltpu.VMEM((R128, K), jnp.int32),
        pltpu.VMEM((K,), jnp.int32),
        pltpu.VMEM((K,), jnp.int32),
        pltpu.VMEM((K, 64), jnp.bfloat16),
        pltpu.VMEM((K, 64), jnp.bfloat16),
        pltpu.VMEM((K, 64), jnp.bfloat16),
        pltpu.VMEM((K, 64), jnp.bfloat16),
        pltpu.VMEM_SHARED((P, 64), jnp.bfloat16),
        pltpu.VMEM_SHARED((P, 64), jnp.bfloat16),
    ] + [pltpu.SemaphoreType.DMA] * 8,
)(_sc_agg128_body)


def _sc_agg16_body(h_hbm, e3_hbm, out_hbm,
                   srcall, dstall, xsrc, xdst,
                   rows0, rows1, rows2, rows3, acc,
                   gs0, gs1, gs2, gs3, ss0, ss1, ss2, ss3):
    c = lax.axis_index("c")
    s = lax.axis_index("s")
    wid = c * NS + s

    pltpu.sync_copy(e3_hbm.at[0, pl.ds(wid * R16, R16)], srcall)
    pltpu.sync_copy(e3_hbm.at[1, pl.ds(wid * R16, R16)], dstall)

    _zero_rows_f32(rows0, K, 16)
    for t in range(RPT // K):
        pltpu.sync_copy(rows0, acc.at[pl.ds(s * RPT + t * K, K)])
    plsc.subcore_barrier()

    def issue(b, buf, sem):
        pltpu.async_copy(h_hbm.at[srcall.at[b]], buf, sem)

    RMAIN = 76
    _ring(issue, h_hbm, srcall, dstall, [rows0, rows1, rows2, rows3], acc,
          [gs0, gs1, gs2, gs3], [ss0, ss1, ss2, ss3], RMAIN)

    for b in (76, 77):
        pltpu.async_copy(h_hbm.at[srcall.at[b]], rows0, gs0).wait()
        pltpu.sync_copy(rows0, acc.at[dstall.at[b]], add=True)

    @pl.when(wid < 4)
    def _():
        pltpu.sync_copy(e3_hbm.at[0, ER - 4 + wid], xsrc)
        pltpu.sync_copy(e3_hbm.at[1, ER - 4 + wid], xdst)
        pltpu.async_copy(h_hbm.at[xsrc], rows0, gs0).wait()
        pltpu.sync_copy(rows0, acc.at[xdst], add=True)

    plsc.subcore_barrier()
    pltpu.sync_copy(acc.at[pl.ds(s * RPT, RPT)],
                    out_hbm.at[pl.ds(c * P + s * RPT, RPT)])


_agg16_call = functools.partial(
    pl.kernel,
    out_type=jax.ShapeDtypeStruct((2 * P, 16), jnp.float32),
    mesh=_MESH,
    compiler_params=pltpu.CompilerParams(use_tc_tiling_on_sc=False),
    scratch_types=[
        pltpu.VMEM((R16, K), jnp.int32),
        pltpu.VMEM((R16, K), jnp.int32),
        pltpu.VMEM((K,), jnp.int32),
        pltpu.VMEM((K,), jnp.int32),
        pltpu.VMEM((K, 16), jnp.float32),
        pltpu.VMEM((K, 16), jnp.float32),
        pltpu.VMEM((K, 16), jnp.float32),
        pltpu.VMEM((K, 16), jnp.float32),
        pltpu.VMEM_SHARED((P, 16), jnp.float32),
    ] + [pltpu.SemaphoreType.DMA] * 8,
)(_sc_agg16_body)


BLK = 1024
GRID = P // BLK
BLKO = 2048
GRIDO = P // BLKO


def _pk_to_col(d, nrows):
    g = nrows // K
    x = jnp.broadcast_to(d[:, None, :], (g, K, K)).reshape(nrows, K)
    l = lax.broadcasted_iota(jnp.int32, (nrows, K), 1)
    n = lax.broadcasted_iota(jnp.int32, (nrows, K), 0)
    return jnp.sum(jnp.where(l == n % K, x, 0.0), axis=1, keepdims=True)


def _tc_mm1a_body(x_ref, w1_ref, h_ref):
    i = pl.program_id(0)
    h = jnp.dot(x_ref[...], w1_ref[...], preferred_element_type=jnp.float32)
    row = i * BLK + lax.broadcasted_iota(jnp.int32, (BLK, 128), 0)
    h_ref[...] = jnp.where(row < N, h, 0.0)


def _tc_mm1a(x, W1):
    return pl.pallas_call(
        _tc_mm1a_body,
        grid=(GRID,),
        in_specs=[
            pl.BlockSpec((BLK, 128), lambda i: (i, 0)),
            pl.BlockSpec((128, 128), lambda i: (0, 0)),
        ],
        out_specs=pl.BlockSpec((BLK, 128), lambda i: (i, 0)),
        out_shape=jax.ShapeDtypeStruct((P, 128), jnp.float32),
    )(x, W1)


def _tc_mm1b_body(h_ref, d0_ref, d1_ref, hlo_ref, hhi_ref, dis_ref):
    dis = lax.rsqrt(d0_ref[...] + d1_ref[...] + 1.0)
    dis_ref[...] = dis
    h = h_ref[...] * _pk_to_col(dis, BLK)
    hb = h.astype(jnp.bfloat16)
    hlo_ref[...] = hb[:, :64]
    hhi_ref[...] = hb[:, 64:]


def _tc_mm1b(hraw, deg_pk):
    return pl.pallas_call(
        _tc_mm1b_body,
        grid=(GRID,),
        in_specs=[
            pl.BlockSpec((BLK, 128), lambda i: (i, 0)),
            pl.BlockSpec((BLK // K, K), lambda i: (i, 0)),
            pl.BlockSpec((BLK // K, K), lambda i: (i + GRID, 0)),
        ],
        out_specs=[
            pl.BlockSpec((BLK, 64), lambda i: (i, 0)),
            pl.BlockSpec((BLK, 64), lambda i: (i, 0)),
            pl.BlockSpec((BLK // K, K), lambda i: (i, 0)),
        ],
        out_shape=[
            jax.ShapeDtypeStruct((P, 64), jnp.bfloat16),
            jax.ShapeDtypeStruct((P, 64), jnp.bfloat16),
            jax.ShapeDtypeStruct((P // K, K), jnp.float32),
        ],
    )(hraw, deg_pk, deg_pk)


def _tc_mm2_body(alo_ref, ahi_ref, hlo_ref, hhi_ref, dis_ref, w2_ref, b1_ref,
                 out_ref):
    dis1 = _pk_to_col(dis_ref[...], BLK)
    b1 = b1_ref[...]
    w2 = w2_ref[...]
    pre_lo = ((alo_ref[...].astype(jnp.float32)
               + hlo_ref[...].astype(jnp.float32)) * dis1 + b1[:, :64])
    pre_hi = ((ahi_ref[...].astype(jnp.float32)
               + hhi_ref[...].astype(jnp.float32)) * dis1 + b1[:, 64:])
    o_lo = jnp.maximum(pre_lo, 0.0)
    o_hi = jnp.maximum(pre_hi, 0.0)
    h2 = (jnp.dot(o_lo, w2[:64], preferred_element_type=jnp.float32)
          + jnp.dot(o_hi, w2[64:], preferred_element_type=jnp.float32))
    out_ref[...] = h2 * dis1


def _tc_mm2(a128, hlo, hhi, dis, W2, b1):
    return pl.pallas_call(
        _tc_mm2_body,
        grid=(GRID,),
        in_specs=[
            pl.BlockSpec((BLK, 64), lambda i: (i, 0)),
            pl.BlockSpec((BLK, 64), lambda i: (i + GRID, 0)),
            pl.BlockSpec((BLK, 64), lambda i: (i, 0)),
            pl.BlockSpec((BLK, 64), lambda i: (i, 0)),
            pl.BlockSpec((BLK // K, K), lambda i: (i, 0)),
            pl.BlockSpec((128, 16), lambda i: (0, 0)),
            pl.BlockSpec((1, 128), lambda i: (0, 0)),
        ],
        out_specs=pl.BlockSpec((BLK, 16), lambda i: (i, 0)),
        out_shape=jax.ShapeDtypeStruct((P, 16), jnp.float32),
    )(a128, a128, hlo, hhi, dis, W2, b1)


def _tc_out_body(p0_ref, p1_ref, h2_ref, dis_ref, b2_ref, out_ref):
    dis1 = _pk_to_col(dis_ref[...], BLKO)
    out_ref[...] = (p0_ref[...] + p1_ref[...] + h2_ref[...]) * dis1 + b2_ref[...]


def _tc_out(acc2, h2p, dis, b2):
    return pl.pallas_call(
        _tc_out_body,
        grid=(GRIDO,),
        in_specs=[
            pl.BlockSpec((BLKO, 16), lambda i: (i, 0)),
            pl.BlockSpec((BLKO, 16), lambda i: (i + GRIDO, 0)),
            pl.BlockSpec((BLKO, 16), lambda i: (i, 0)),
            pl.BlockSpec((BLKO // K, K), lambda i: (i, 0)),
            pl.BlockSpec((1, 16), lambda i: (0, 0)),
        ],
        out_specs=pl.BlockSpec((BLKO, 16), lambda i: (i, 0)),
        out_shape=jax.ShapeDtypeStruct((N, 16), jnp.float32),
    )(acc2, acc2, h2p, dis, b2)


@jax.jit
def kernel(x, edge_index, W1, b1, W2, b2):
    e3 = edge_index.reshape(2, ER, K)

    deg_pk = _deg_call(e3)
    hraw = _tc_mm1a(x, W1)
    hlo, hhi, dis = _tc_mm1b(hraw, deg_pk)

    acc1 = _agg128_call(hlo, hhi, e3)
    h2p = _tc_mm2(acc1, hlo, hhi, dis, W2, b1.reshape(1, 128))

    acc2 = _agg16_call(h2p, e3)
    return _tc_out(acc2, h2p, dis, b2.reshape(1, 16))


# speedup vs baseline: 1.0484x; 1.0263x over previous
import functools

import jax
import jax.numpy as jnp
from jax import lax
from jax.experimental import pallas as pl
from jax.experimental.pallas import tpu as pltpu
from jax.experimental.pallas import tpu_sc as plsc

N = 10000
E = 320000
P = 10240
NC, NS = 2, 16
NW = NC * NS
K = 128
ER = E // K
RPT = P // NS
R16 = 78
R128 = 156

_MESH = plsc.VectorSubcoreMesh(core_axis_name="c", subcore_axis_name="s",
                               num_cores=NC, num_subcores=NS)


def _zero_rows_f32(ref, nrows, ncols):
    z = jnp.zeros((16,), jnp.float32)

    @pl.loop(0, ncols // 16)
    def _(j):
        @pl.loop(0, nrows)
        def _(r):
            ref[r, pl.ds(j * 16, 16)] = z


def _zero_rows_bf16(ref, nrows, ncols):
    z = jnp.zeros((32,), jnp.bfloat16)

    @pl.loop(0, ncols // 32)
    def _(j):
        @pl.loop(0, nrows)
        def _(r):
            ref[r, pl.ds(j * 32, 32)] = z


def _ring(issue_gather, wait_table, srcall, dstall, rows, acc,
          gsems, ssems, nb):
    L = 4
    for j in range(L - 1):
        issue_gather(j, rows[j], gsems[j])

    @pl.loop(0, nb, step=L)
    def _(base):
        for u in range(L):
            b = base + u
            j = u
            pltpu.make_async_copy(wait_table.at[srcall.at[b]], rows[j],
                                  gsems[j]).wait()
            pltpu.async_copy(rows[j], acc.at[dstall.at[b]], ssems[j],
                             add=True)
            jp = (j + L - 1) % L

            @pl.when(b >= 1)
            def _():
                pltpu.make_async_copy(rows[jp], acc.at[dstall.at[b - 1]],
                                      ssems[jp]).wait()

            @pl.when(b + L - 1 < nb)
            def _():
                issue_gather(b + L - 1, rows[jp], gsems[jp])

    pltpu.make_async_copy(rows[(nb - 1) % L], acc.at[dstall.at[nb - 1]],
                          ssems[(nb - 1) % L]).wait()


def _sc_deg_body(e3_hbm, out_hbm, dstall, xdst, ones, zbuf, dbuf, pbuf, acc):
    c = lax.axis_index("c")
    s = lax.axis_index("s")
    wid = c * NS + s

    pltpu.sync_copy(e3_hbm.at[1, pl.ds(wid * R16, R16)], dstall)

    @pl.loop(0, K)
    def _(r):
        ones[r, :] = jnp.ones((16,), jnp.float32)

    _zero_rows_f32(zbuf, K, 16)
    for t in range(RPT // K):
        pltpu.sync_copy(zbuf, acc.at[pl.ds(s * RPT + t * K, K)])
    plsc.subcore_barrier()

    @pl.loop(0, R16)
    def _(b):
        pltpu.sync_copy(ones, acc.at[dstall.at[b]], add=True)

    @pl.when(wid < 4)
    def _():
        pltpu.sync_copy(e3_hbm.at[1, ER - 4 + wid], xdst)
        pltpu.sync_copy(ones, acc.at[xdst], add=True)

    plsc.subcore_barrier()
    pltpu.sync_copy(acc.at[pl.ds(s * RPT, RPT)], dbuf)
    lane = lax.iota(jnp.int32, 16)
    for g in range(RPT // 16):
        d = jnp.zeros((16,), jnp.float32)
        for r in range(16):
            d = jnp.where(lane == r, dbuf[16 * g + r, :], d)
        pbuf[g // 8, pl.ds(16 * (g % 8), 16)] = d
    pltpu.sync_copy(
        pbuf, out_hbm.at[pl.ds((c * NS + s) * (RPT // K), RPT // K)])


_deg_call = functools.partial(
    pl.kernel,
    out_type=jax.ShapeDtypeStruct((2 * P // K, K), jnp.float32),
    mesh=_MESH,
    compiler_params=pltpu.CompilerParams(use_tc_tiling_on_sc=False),
    scratch_types=[
        pltpu.VMEM((R16, K), jnp.int32),
        pltpu.VMEM((K,), jnp.int32),
        pltpu.VMEM((K, 16), jnp.float32),
        pltpu.VMEM((K, 16), jnp.float32),
        pltpu.VMEM((RPT, 16), jnp.float32),
        pltpu.VMEM((RPT // K, K), jnp.float32),
        pltpu.VMEM_SHARED((P, 16), jnp.float32),
    ],
)(_sc_deg_body)


def _sc_agg128_body(hlo_hbm, hhi_hbm, e3_hbm, out_hbm,
                    srcall, dstall, xsrc, xdst,
                    rows0, rows1, rows2, rows3, acc,
                    gs0, gs1, gs2, gs3, ss0, ss1, ss2, ss3):
    c = lax.axis_index("c")
    s = lax.axis_index("s")

    pltpu.sync_copy(e3_hbm.at[0, pl.ds(s * R128, R128)], srcall)
    pltpu.sync_copy(e3_hbm.at[1, pl.ds(s * R128, R128)], dstall)

    @pl.when(c == 0)
    def _():
        pltpu.sync_copy(hlo_hbm.at[pl.ds(s * RPT, RPT)],
                        acc.at[pl.ds(s * RPT, RPT)])

    @pl.when(c == 1)
    def _():
        pltpu.sync_copy(hhi_hbm.at[pl.ds(s * RPT, RPT)],
                        acc.at[pl.ds(s * RPT, RPT)])

    plsc.subcore_barrier()

    def issue(b, buf, sem):
        @pl.when(c == 0)
        def _():
            pltpu.async_copy(hlo_hbm.at[srcall.at[b]], buf, sem)

        @pl.when(c == 1)
        def _():
            pltpu.async_copy(hhi_hbm.at[srcall.at[b]], buf, sem)

    _ring(issue, hlo_hbm, srcall, dstall, [rows0, rows1, rows2, rows3], acc,
          [gs0, gs1, gs2, gs3], [ss0, ss1, ss2, ss3], R128)

    @pl.when(s < 4)
    def _():
        pltpu.sync_copy(e3_hbm.at[0, ER - 4 + s], xsrc)
        pltpu.sync_copy(e3_hbm.at[1, ER - 4 + s], xdst)

        @pl.when(c == 0)
        def _():
            pltpu.async_copy(hlo_hbm.at[xsrc], rows0, gs0).wait()

        @pl.when(c == 1)
        def _():
            pltpu.async_copy(hhi_hbm.at[xsrc], rows0, gs0).wait()

        pltpu.sync_copy(rows0, acc.at[xdst], add=True)

    plsc.subcore_barrier()
    pltpu.sync_copy(acc.at[pl.ds(s * RPT, RPT)],
                    out_hbm.at[pl.ds(c * P + s * RPT, RPT)])


_agg128_call = functools.partial(
    pl.kernel,
    out_type=jax.ShapeDtypeStruct((2 * P, 64), jnp.bfloat16),
    mesh=_MESH,
    compiler_params=pltpu.CompilerParams(use_tc_tiling_on_sc=False),
    scratch_types=[
        pltpu.VMEM((R128, K), jnp.int32),
        pltpu.VMEM((R128, K), jnp.int32),
        pltpu.VMEM((K,), jnp.int32),
        pltpu.VMEM((K,), jnp.int32),
        pltpu.VMEM((K, 64), jnp.bfloat16),
        pltpu.VMEM((K, 64), jnp.bfloat16),
        pltpu.VMEM((K, 64), jnp.bfloat16),
        pltpu.VMEM((K, 64), jnp.bfloat16),
        pltpu.VMEM_SHARED((P, 64), jnp.bfloat16),
    ] + [pltpu.SemaphoreType.DMA] * 8,
)(_sc_agg128_body)


def _sc_agg16_body(h_hbm, e3_hbm, out_hbm,
                   srcall, dstall, xsrc, xdst,
                   rows0, rows1, rows2, rows3, acc,
                   gs0, gs1, gs2, gs3, ss0, ss1, ss2, ss3):
    c = lax.axis_index("c")
    s = lax.axis_index("s")
    wid = c * NS + s

    pltpu.sync_copy(e3_hbm.at[0, pl.ds(wid * R16, R16)], srcall)
    pltpu.sync_copy(e3_hbm.at[1, pl.ds(wid * R16, R16)], dstall)

    @pl.when(c == 0)
    def _():
        pltpu.sync_copy(h_hbm.at[pl.ds(s * RPT, RPT)],
                        acc.at[pl.ds(s * RPT, RPT)])

    @pl.when(c == 1)
    def _():
        _zero_rows_f32(rows0, K, 16)
        for t in range(RPT // K):
            pltpu.sync_copy(rows0, acc.at[pl.ds(s * RPT + t * K, K)])

    plsc.subcore_barrier()

    def issue(b, buf, sem):
        pltpu.async_copy(h_hbm.at[srcall.at[b]], buf, sem)

    RMAIN = 76
    _ring(issue, h_hbm, srcall, dstall, [rows0, rows1, rows2, rows3], acc,
          [gs0, gs1, gs2, gs3], [ss0, ss1, ss2, ss3], RMAIN)

    for b in (76, 77):
        pltpu.async_copy(h_hbm.at[srcall.at[b]], rows0, gs0).wait()
        pltpu.sync_copy(rows0, acc.at[dstall.at[b]], add=True)

    @pl.when(wid < 4)
    def _():
        pltpu.sync_copy(e3_hbm.at[0, ER - 4 + wid], xsrc)
        pltpu.sync_copy(e3_hbm.at[1, ER - 4 + wid], xdst)
        pltpu.async_copy(h_hbm.at[xsrc], rows0, gs0).wait()
        pltpu.sync_copy(rows0, acc.at[xdst], add=True)

    plsc.subcore_barrier()
    pltpu.sync_copy(acc.at[pl.ds(s * RPT, RPT)],
                    out_hbm.at[pl.ds(c * P + s * RPT, RPT)])


_agg16_call = functools.partial(
    pl.kernel,
    out_type=jax.ShapeDtypeStruct((2 * P, 16), jnp.float32),
    mesh=_MESH,
    compiler_params=pltpu.CompilerParams(use_tc_tiling_on_sc=False),
    scratch_types=[
        pltpu.VMEM((R16, K), jnp.int32),
        pltpu.VMEM((R16, K), jnp.int32),
        pltpu.VMEM((K,), jnp.int32),
        pltpu.VMEM((K,), jnp.int32),
        pltpu.VMEM((K, 16), jnp.float32),
        pltpu.VMEM((K, 16), jnp.float32),
        pltpu.VMEM((K, 16), jnp.float32),
        pltpu.VMEM((K, 16), jnp.float32),
        pltpu.VMEM_SHARED((P, 16), jnp.float32),
    ] + [pltpu.SemaphoreType.DMA] * 8,
)(_sc_agg16_body)


BLK = 1024
GRID = P // BLK
BLKO = 2048
GRIDO = P // BLKO


def _pk_to_col(d, nrows):
    g = nrows // K
    x = jnp.broadcast_to(d[:, None, :], (g, K, K)).reshape(nrows, K)
    l = lax.broadcasted_iota(jnp.int32, (nrows, K), 1)
    n = lax.broadcasted_iota(jnp.int32, (nrows, K), 0)
    return jnp.sum(jnp.where(l == n % K, x, 0.0), axis=1, keepdims=True)


def _tc_mm1a_body(x_ref, w1_ref, h_ref):
    i = pl.program_id(0)
    h = jnp.dot(x_ref[...], w1_ref[...], preferred_element_type=jnp.float32)
    row = i * BLK + lax.broadcasted_iota(jnp.int32, (BLK, 128), 0)
    h_ref[...] = jnp.where(row < N, h, 0.0)


def _tc_mm1a(x, W1):
    return pl.pallas_call(
        _tc_mm1a_body,
        grid=(GRID,),
        in_specs=[
            pl.BlockSpec((BLK, 128), lambda i: (i, 0)),
            pl.BlockSpec((128, 128), lambda i: (0, 0)),
        ],
        out_specs=pl.BlockSpec((BLK, 128), lambda i: (i, 0)),
        out_shape=jax.ShapeDtypeStruct((P, 128), jnp.float32),
    )(x, W1)


def _tc_mm1b_body(h_ref, d0_ref, d1_ref, hlo_ref, hhi_ref, dis_ref):
    dis = lax.rsqrt(d0_ref[...] + d1_ref[...] + 1.0)
    dis_ref[...] = dis
    h = h_ref[...] * _pk_to_col(dis, BLK)
    hb = h.astype(jnp.bfloat16)
    hlo_ref[...] = hb[:, :64]
    hhi_ref[...] = hb[:, 64:]


def _tc_mm1b(hraw, deg_pk):
    return pl.pallas_call(
        _tc_mm1b_body,
        grid=(GRID,),
        in_specs=[
            pl.BlockSpec((BLK, 128), lambda i: (i, 0)),
            pl.BlockSpec((BLK // K, K), lambda i: (i, 0)),
            pl.BlockSpec((BLK // K, K), lambda i: (i + GRID, 0)),
        ],
        out_specs=[
            pl.BlockSpec((BLK, 64), lambda i: (i, 0)),
            pl.BlockSpec((BLK, 64), lambda i: (i, 0)),
            pl.BlockSpec((BLK // K, K), lambda i: (i, 0)),
        ],
        out_shape=[
            jax.ShapeDtypeStruct((P, 64), jnp.bfloat16),
            jax.ShapeDtypeStruct((P, 64), jnp.bfloat16),
            jax.ShapeDtypeStruct((P // K, K), jnp.float32),
        ],
    )(hraw, deg_pk, deg_pk)


def _tc_mm2_body(alo_ref, ahi_ref, dis_ref, w2_ref, b1_ref, out_ref):
    dis1 = _pk_to_col(dis_ref[...], BLK)
    b1 = b1_ref[...]
    w2 = w2_ref[...]
    pre_lo = alo_ref[...].astype(jnp.float32) * dis1 + b1[:, :64]
    pre_hi = ahi_ref[...].astype(jnp.float32) * dis1 + b1[:, 64:]
    o_lo = jnp.maximum(pre_lo, 0.0)
    o_hi = jnp.maximum(pre_hi, 0.0)
    h2 = (jnp.dot(o_lo, w2[:64], preferred_element_type=jnp.float32)
          + jnp.dot(o_hi, w2[64:], preferred_element_type=jnp.float32))
    out_ref[...] = h2 * dis1


def _tc_mm2(a128, dis, W2, b1):
    return pl.pallas_call(
        _tc_mm2_body,
        grid=(GRID,),
        in_specs=[
            pl.BlockSpec((BLK, 64), lambda i: (i, 0)),
            pl.BlockSpec((BLK, 64), lambda i: (i + GRID, 0)),
            pl.BlockSpec((BLK // K, K), lambda i: (i, 0)),
            pl.BlockSpec((128, 16), lambda i: (0, 0)),
            pl.BlockSpec((1, 128), lambda i: (0, 0)),
        ],
        out_specs=pl.BlockSpec((BLK, 16), lambda i: (i, 0)),
        out_shape=jax.ShapeDtypeStruct((P, 16), jnp.float32),
    )(a128, a128, dis, W2, b1)


def _tc_out_body(p0_ref, p1_ref, dis_ref, b2_ref, out_ref):
    dis1 = _pk_to_col(dis_ref[...], BLKO)
    out_ref[...] = (p0_ref[...] + p1_ref[...]) * dis1 + b2_ref[...]


def _tc_out(acc2, dis, b2):
    return pl.pallas_call(
        _tc_out_body,
        grid=(GRIDO,),
        in_specs=[
            pl.BlockSpec((BLKO, 16), lambda i: (i, 0)),
            pl.BlockSpec((BLKO, 16), lambda i: (i + GRIDO, 0)),
            pl.BlockSpec((BLKO // K, K), lambda i: (i, 0)),
            pl.BlockSpec((1, 16), lambda i: (0, 0)),
        ],
        out_specs=pl.BlockSpec((BLKO, 16), lambda i: (i, 0)),
        out_shape=jax.ShapeDtypeStruct((N, 16), jnp.float32),
    )(acc2, acc2, dis, b2)


@jax.jit
def kernel(x, edge_index, W1, b1, W2, b2):
    e3 = edge_index.reshape(2, ER, K)

    deg_pk = _deg_call(e3)
    hraw = _tc_mm1a(x, W1)
    hlo, hhi, dis = _tc_mm1b(hraw, deg_pk)

    acc1 = _agg128_call(hlo, hhi, e3)
    h2p = _tc_mm2(acc1, dis, W2, b1.reshape(1, 128))

    acc2 = _agg16_call(h2p, e3)
    return _tc_out(acc2, dis, b2.reshape(1, 16))


# speedup vs baseline: 1.0710x; 1.0216x over previous
import functools

import jax
import jax.numpy as jnp
from jax import lax
from jax.experimental import pallas as pl
from jax.experimental.pallas import tpu as pltpu
from jax.experimental.pallas import tpu_sc as plsc

N = 10000
E = 320000
P = 10240
NC, NS = 2, 16
NW = NC * NS
K = 128
ER = E // K
RPT = P // NS
R16 = 78
R128 = 156

_MESH = plsc.VectorSubcoreMesh(core_axis_name="c", subcore_axis_name="s",
                               num_cores=NC, num_subcores=NS)


def _zero_rows_f32(ref, nrows, ncols):
    z = jnp.zeros((16,), jnp.float32)

    @pl.loop(0, ncols // 16)
    def _(j):
        @pl.loop(0, nrows)
        def _(r):
            ref[r, pl.ds(j * 16, 16)] = z


def _zero_rows_bf16(ref, nrows, ncols):
    z = jnp.zeros((32,), jnp.bfloat16)

    @pl.loop(0, ncols // 32)
    def _(j):
        @pl.loop(0, nrows)
        def _(r):
            ref[r, pl.ds(j * 32, 32)] = z


def _ring(issue_gather, wait_table, srcall, dstall, rows, acc,
          gsems, ssems, nb):
    L = 4
    for j in range(L - 1):
        issue_gather(j, rows[j], gsems[j])

    @pl.loop(0, nb, step=L)
    def _(base):
        for u in range(L):
            b = base + u
            j = u
            pltpu.make_async_copy(wait_table.at[srcall.at[b]], rows[j],
                                  gsems[j]).wait()
            pltpu.async_copy(rows[j], acc.at[dstall.at[b]], ssems[j],
                             add=True)
            jp = (j + L - 1) % L

            @pl.when(b >= 1)
            def _():
                pltpu.make_async_copy(rows[jp], acc.at[dstall.at[b - 1]],
                                      ssems[jp]).wait()

            @pl.when(b + L - 1 < nb)
            def _():
                issue_gather(b + L - 1, rows[jp], gsems[jp])

    pltpu.make_async_copy(rows[(nb - 1) % L], acc.at[dstall.at[nb - 1]],
                          ssems[(nb - 1) % L]).wait()


def _sc_deg_body(e3_hbm, out_hbm, dstall, xdst, ones, zbuf, dbuf, pbuf, acc,
                 ssem):
    c = lax.axis_index("c")
    s = lax.axis_index("s")
    wid = c * NS + s

    pltpu.sync_copy(e3_hbm.at[1, pl.ds(wid * R16, R16)], dstall)

    @pl.loop(0, K)
    def _(r):
        ones[r, :] = jnp.ones((16,), jnp.float32)

    _zero_rows_f32(zbuf, K, 16)
    for t in range(RPT // K):
        pltpu.sync_copy(zbuf, acc.at[pl.ds(s * RPT + t * K, K)])
    plsc.subcore_barrier()

    @pl.loop(0, R16)
    def _(b):
        pltpu.async_copy(ones, acc.at[dstall.at[b]], ssem, add=True)

    @pl.when(wid < 4)
    def _():
        pltpu.sync_copy(e3_hbm.at[1, ER - 4 + wid], xdst)
        pltpu.sync_copy(ones, acc.at[xdst], add=True)

    @pl.loop(0, R16)
    def _(b):
        pltpu.make_async_copy(ones, acc.at[dstall.at[0]], ssem).wait()

    plsc.subcore_barrier()
    pltpu.sync_copy(acc.at[pl.ds(s * RPT, RPT)], dbuf)
    lane = lax.iota(jnp.int32, 16)
    for g in range(RPT // 16):
        d = jnp.zeros((16,), jnp.float32)
        for r in range(16):
            d = jnp.where(lane == r, dbuf[16 * g + r, :], d)
        pbuf[g // 8, pl.ds(16 * (g % 8), 16)] = d
    pltpu.sync_copy(
        pbuf, out_hbm.at[pl.ds((c * NS + s) * (RPT // K), RPT // K)])


_deg_call = functools.partial(
    pl.kernel,
    out_type=jax.ShapeDtypeStruct((2 * P // K, K), jnp.float32),
    mesh=_MESH,
    compiler_params=pltpu.CompilerParams(use_tc_tiling_on_sc=False),
    scratch_types=[
        pltpu.VMEM((R16, K), jnp.int32),
        pltpu.VMEM((K,), jnp.int32),
        pltpu.VMEM((K, 16), jnp.float32),
        pltpu.VMEM((K, 16), jnp.float32),
        pltpu.VMEM((RPT, 16), jnp.float32),
        pltpu.VMEM((RPT // K, K), jnp.float32),
        pltpu.VMEM_SHARED((P, 16), jnp.float32),
        pltpu.SemaphoreType.DMA,
    ],
)(_sc_deg_body)


def _sc_agg128_body(hlo_hbm, hhi_hbm, e3_hbm, out_hbm,
                    srcall, dstall, xsrc, xdst,
                    rows0, rows1, rows2, rows3, acc,
                    gs0, gs1, gs2, gs3, ss0, ss1, ss2, ss3):
    c = lax.axis_index("c")
    s = lax.axis_index("s")

    pltpu.sync_copy(e3_hbm.at[0, pl.ds(s * R128, R128)], srcall)
    pltpu.sync_copy(e3_hbm.at[1, pl.ds(s * R128, R128)], dstall)

    @pl.when(c == 0)
    def _():
        pltpu.sync_copy(hlo_hbm.at[pl.ds(s * RPT, RPT)],
                        acc.at[pl.ds(s * RPT, RPT)])

    @pl.when(c == 1)
    def _():
        pltpu.sync_copy(hhi_hbm.at[pl.ds(s * RPT, RPT)],
                        acc.at[pl.ds(s * RPT, RPT)])

    plsc.subcore_barrier()

    def issue(b, buf, sem):
        @pl.when(c == 0)
        def _():
            pltpu.async_copy(hlo_hbm.at[srcall.at[b]], buf, sem)

        @pl.when(c == 1)
        def _():
            pltpu.async_copy(hhi_hbm.at[srcall.at[b]], buf, sem)

    _ring(issue, hlo_hbm, srcall, dstall, [rows0, rows1, rows2, rows3], acc,
          [gs0, gs1, gs2, gs3], [ss0, ss1, ss2, ss3], R128)

    @pl.when(s < 4)
    def _():
        pltpu.sync_copy(e3_hbm.at[0, ER - 4 + s], xsrc)
        pltpu.sync_copy(e3_hbm.at[1, ER - 4 + s], xdst)

        @pl.when(c == 0)
        def _():
            pltpu.async_copy(hlo_hbm.at[xsrc], rows0, gs0).wait()

        @pl.when(c == 1)
        def _():
            pltpu.async_copy(hhi_hbm.at[xsrc], rows0, gs0).wait()

        pltpu.sync_copy(rows0, acc.at[xdst], add=True)

    plsc.subcore_barrier()
    pltpu.sync_copy(acc.at[pl.ds(s * RPT, RPT)],
                    out_hbm.at[pl.ds(c * P + s * RPT, RPT)])


_agg128_call = functools.partial(
    pl.kernel,
    out_type=jax.ShapeDtypeStruct((2 * P, 64), jnp.bfloat16),
    mesh=_MESH,
    compiler_params=pltpu.CompilerParams(use_tc_tiling_on_sc=False),
    scratch_types=[
        pltpu.VMEM((R128, K), jnp.int32),
        pltpu.VMEM((R128, K), jnp.int32),
        pltpu.VMEM((K,), jnp.int32),
        pltpu.VMEM((K,), jnp.int32),
        pltpu.VMEM((K, 64), jnp.bfloat16),
        pltpu.VMEM((K, 64), jnp.bfloat16),
        pltpu.VMEM((K, 64), jnp.bfloat16),
        pltpu.VMEM((K, 64), jnp.bfloat16),
        pltpu.VMEM_SHARED((P, 64), jnp.bfloat16),
    ] + [pltpu.SemaphoreType.DMA] * 8,
)(_sc_agg128_body)


def _sc_agg16_body(h_hbm, e3_hbm, out_hbm,
                   srcall, dstall, xsrc, xdst,
                   rows0, rows1, rows2, rows3, acc,
                   gs0, gs1, gs2, gs3, ss0, ss1, ss2, ss3):
    c = lax.axis_index("c")
    s = lax.axis_index("s")
    wid = c * NS + s

    pltpu.sync_copy(e3_hbm.at[0, pl.ds(wid * R16, R16)], srcall)
    pltpu.sync_copy(e3_hbm.at[1, pl.ds(wid * R16, R16)], dstall)

    @pl.when(c == 0)
    def _():
        pltpu.sync_copy(h_hbm.at[pl.ds(s * RPT, RPT)],
                        acc.at[pl.ds(s * RPT, RPT)])

    @pl.when(c == 1)
    def _():
        _zero_rows_f32(rows0, K, 16)
        for t in range(RPT // K):
            pltpu.sync_copy(rows0, acc.at[pl.ds(s * RPT + t * K, K)])

    plsc.subcore_barrier()

    def issue(b, buf, sem):
        pltpu.async_copy(h_hbm.at[srcall.at[b]], buf, sem)

    RMAIN = 76
    _ring(issue, h_hbm, srcall, dstall, [rows0, rows1, rows2, rows3], acc,
          [gs0, gs1, gs2, gs3], [ss0, ss1, ss2, ss3], RMAIN)

    for b in (76, 77):
        pltpu.async_copy(h_hbm.at[srcall.at[b]], rows0, gs0).wait()
        pltpu.sync_copy(rows0, acc.at[dstall.at[b]], add=True)

    @pl.when(wid < 4)
    def _():
        pltpu.sync_copy(e3_hbm.at[0, ER - 4 + wid], xsrc)
        pltpu.sync_copy(e3_hbm.at[1, ER - 4 + wid], xdst)
        pltpu.async_copy(h_hbm.at[xsrc], rows0, gs0).wait()
        pltpu.sync_copy(rows0, acc.at[xdst], add=True)

    plsc.subcore_barrier()
    pltpu.sync_copy(acc.at[pl.ds(s * RPT, RPT)],
                    out_hbm.at[pl.ds(c * P + s * RPT, RPT)])


_agg16_call = functools.partial(
    pl.kernel,
    out_type=jax.ShapeDtypeStruct((2 * P, 16), jnp.float32),
    mesh=_MESH,
    compiler_params=pltpu.CompilerParams(use_tc_tiling_on_sc=False),
    scratch_types=[
        pltpu.VMEM((R16, K), jnp.int32),
        pltpu.VMEM((R16, K), jnp.int32),
        pltpu.VMEM((K,), jnp.int32),
        pltpu.VMEM((K,), jnp.int32),
        pltpu.VMEM((K, 16), jnp.float32),
        pltpu.VMEM((K, 16), jnp.float32),
        pltpu.VMEM((K, 16), jnp.float32),
        pltpu.VMEM((K, 16), jnp.float32),
        pltpu.VMEM_SHARED((P, 16), jnp.float32),
    ] + [pltpu.SemaphoreType.DMA] * 8,
)(_sc_agg16_body)


BLK = 1024
GRID = P // BLK
BLKO = 2048
GRIDO = P // BLKO


def _pk_to_col(d, nrows):
    g = nrows // K
    x = jnp.broadcast_to(d[:, None, :], (g, K, K)).reshape(nrows, K)
    l = lax.broadcasted_iota(jnp.int32, (nrows, K), 1)
    n = lax.broadcasted_iota(jnp.int32, (nrows, K), 0)
    return jnp.sum(jnp.where(l == n % K, x, 0.0), axis=1, keepdims=True)


def _tc_mm1a_body(x_ref, w1_ref, h_ref):
    i = pl.program_id(0)
    h = jnp.dot(x_ref[...], w1_ref[...], preferred_element_type=jnp.float32)
    row = i * BLK + lax.broadcasted_iota(jnp.int32, (BLK, 128), 0)
    h_ref[...] = jnp.where(row < N, h, 0.0)


def _tc_mm1a(x, W1):
    return pl.pallas_call(
        _tc_mm1a_body,
        grid=(GRID,),
        in_specs=[
            pl.BlockSpec((BLK, 128), lambda i: (i, 0)),
            pl.BlockSpec((128, 128), lambda i: (0, 0)),
        ],
        out_specs=pl.BlockSpec((BLK, 128), lambda i: (i, 0)),
        out_shape=jax.ShapeDtypeStruct((P, 128), jnp.float32),
    )(x, W1)


def _tc_mm1b_body(h_ref, d0_ref, d1_ref, hlo_ref, hhi_ref, dis_ref):
    dis = lax.rsqrt(d0_ref[...] + d1_ref[...] + 1.0)
    dis_ref[...] = dis
    h = h_ref[...] * _pk_to_col(dis, BLK)
    hb = h.astype(jnp.bfloat16)
    hlo_ref[...] = hb[:, :64]
    hhi_ref[...] = hb[:, 64:]


def _tc_mm1b(hraw, deg_pk):
    return pl.pallas_call(
        _tc_mm1b_body,
        grid=(GRID,),
        in_specs=[
            pl.BlockSpec((BLK, 128), lambda i: (i, 0)),
            pl.BlockSpec((BLK // K, K), lambda i: (i, 0)),
            pl.BlockSpec((BLK // K, K), lambda i: (i + GRID, 0)),
        ],
        out_specs=[
            pl.BlockSpec((BLK, 64), lambda i: (i, 0)),
            pl.BlockSpec((BLK, 64), lambda i: (i, 0)),
            pl.BlockSpec((BLK // K, K), lambda i: (i, 0)),
        ],
        out_shape=[
            jax.ShapeDtypeStruct((P, 64), jnp.bfloat16),
            jax.ShapeDtypeStruct((P, 64), jnp.bfloat16),
            jax.ShapeDtypeStruct((P // K, K), jnp.float32),
        ],
    )(hraw, deg_pk, deg_pk)


def _tc_mm2_body(alo_ref, ahi_ref, dis_ref, w2_ref, b1_ref, out_ref):
    dis1 = _pk_to_col(dis_ref[...], BLK)
    b1 = b1_ref[...]
    w2 = w2_ref[...]
    pre_lo = alo_ref[...].astype(jnp.float32) * dis1 + b1[:, :64]
    pre_hi = ahi_ref[...].astype(jnp.float32) * dis1 + b1[:, 64:]
    o_lo = jnp.maximum(pre_lo, 0.0)
    o_hi = jnp.maximum(pre_hi, 0.0)
    h2 = (jnp.dot(o_lo, w2[:64], preferred_element_type=jnp.float32)
          + jnp.dot(o_hi, w2[64:], preferred_element_type=jnp.float32))
    out_ref[...] = h2 * dis1


def _tc_mm2(a128, dis, W2, b1):
    return pl.pallas_call(
        _tc_mm2_body,
        grid=(GRID,),
        in_specs=[
            pl.BlockSpec((BLK, 64), lambda i: (i, 0)),
            pl.BlockSpec((BLK, 64), lambda i: (i + GRID, 0)),
            pl.BlockSpec((BLK // K, K), lambda i: (i, 0)),
            pl.BlockSpec((128, 16), lambda i: (0, 0)),
            pl.BlockSpec((1, 128), lambda i: (0, 0)),
        ],
        out_specs=pl.BlockSpec((BLK, 16), lambda i: (i, 0)),
        out_shape=jax.ShapeDtypeStruct((P, 16), jnp.float32),
    )(a128, a128, dis, W2, b1)


def _tc_out_body(p0_ref, p1_ref, dis_ref, b2_ref, out_ref):
    dis1 = _pk_to_col(dis_ref[...], BLKO)
    out_ref[...] = (p0_ref[...] + p1_ref[...]) * dis1 + b2_ref[...]


def _tc_out(acc2, dis, b2):
    return pl.pallas_call(
        _tc_out_body,
        grid=(GRIDO,),
        in_specs=[
            pl.BlockSpec((BLKO, 16), lambda i: (i, 0)),
            pl.BlockSpec((BLKO, 16), lambda i: (i + GRIDO, 0)),
            pl.BlockSpec((BLKO // K, K), lambda i: (i, 0)),
            pl.BlockSpec((1, 16), lambda i: (0, 0)),
        ],
        out_specs=pl.BlockSpec((BLKO, 16), lambda i: (i, 0)),
        out_shape=jax.ShapeDtypeStruct((N, 16), jnp.float32),
    )(acc2, acc2, dis, b2)


@jax.jit
def kernel(x, edge_index, W1, b1, W2, b2):
    e3 = edge_index.reshape(2, ER, K)

    deg_pk = _deg_call(e3)
    hraw = _tc_mm1a(x, W1)
    hlo, hhi, dis = _tc_mm1b(hraw, deg_pk)

    acc1 = _agg128_call(hlo, hhi, e3)
    h2p = _tc_mm2(acc1, dis, W2, b1.reshape(1, 128))

    acc2 = _agg16_call(h2p, e3)
    return _tc_out(acc2, dis, b2.reshape(1, 16))


# speedup vs baseline: 1.0714x; 1.0003x over previous
import functools

import jax
import jax.numpy as jnp
from jax import lax
from jax.experimental import pallas as pl
from jax.experimental.pallas import tpu as pltpu
from jax.experimental.pallas import tpu_sc as plsc

N = 10000
E = 320000
P = 10240
NC, NS = 2, 16
NW = NC * NS
K = 128
ER = E // K
RPT = P // NS
R16 = 78
R128 = 156

_MESH = plsc.VectorSubcoreMesh(core_axis_name="c", subcore_axis_name="s",
                               num_cores=NC, num_subcores=NS)


def _zero_rows_f32(ref, nrows, ncols):
    z = jnp.zeros((16,), jnp.float32)

    @pl.loop(0, ncols // 16)
    def _(j):
        @pl.loop(0, nrows)
        def _(r):
            ref[r, pl.ds(j * 16, 16)] = z


def _ring(issue_gather, wait_table, srcall, dstall, rows, acc,
          gsems, ssems, nb):
    L = 4
    for j in range(L - 1):
        issue_gather(j, rows[j], gsems[j])

    @pl.loop(0, nb, step=L)
    def _(base):
        for u in range(L):
            b = base + u
            j = u
            pltpu.make_async_copy(wait_table.at[srcall.at[b]], rows[j],
                                  gsems[j]).wait()
            pltpu.async_copy(rows[j], acc.at[dstall.at[b]], ssems[j],
                             add=True)
            jp = (j + L - 1) % L

            @pl.when(b >= 1)
            def _():
                pltpu.make_async_copy(rows[jp], acc.at[dstall.at[b - 1]],
                                      ssems[jp]).wait()

            @pl.when(b + L - 1 < nb)
            def _():
                issue_gather(b + L - 1, rows[jp], gsems[jp])

    pltpu.make_async_copy(rows[(nb - 1) % L], acc.at[dstall.at[nb - 1]],
                          ssems[(nb - 1) % L]).wait()


def _sc_deg_body(e3_hbm, out_hbm, dstall, xdst, ones, zbuf, dbuf, pbuf, acc,
                 ssem):
    c = lax.axis_index("c")
    s = lax.axis_index("s")
    wid = c * NS + s

    pltpu.sync_copy(e3_hbm.at[1, pl.ds(wid * R16, R16)], dstall)

    @pl.loop(0, K)
    def _(r):
        ones[r, :] = jnp.ones((16,), jnp.float32)

    _zero_rows_f32(zbuf, K, 16)
    for t in range(RPT // K):
        pltpu.sync_copy(zbuf, acc.at[pl.ds(s * RPT + t * K, K)])
    plsc.subcore_barrier()

    @pl.loop(0, R16)
    def _(b):
        pltpu.async_copy(ones, acc.at[dstall.at[b]], ssem, add=True)

    @pl.when(wid < 4)
    def _():
        pltpu.sync_copy(e3_hbm.at[1, ER - 4 + wid], xdst)
        pltpu.sync_copy(ones, acc.at[xdst], add=True)

    @pl.loop(0, R16)
    def _(b):
        pltpu.make_async_copy(ones, acc.at[dstall.at[0]], ssem).wait()

    plsc.subcore_barrier()
    pltpu.sync_copy(acc.at[pl.ds(s * RPT, RPT)], dbuf)
    lane = lax.iota(jnp.int32, 16)
    for g in range(RPT // 16):
        d = jnp.zeros((16,), jnp.float32)
        for r in range(16):
            d = jnp.where(lane == r, dbuf[16 * g + r, :], d)
        pbuf[g // 8, pl.ds(16 * (g % 8), 16)] = d
    pltpu.sync_copy(
        pbuf, out_hbm.at[pl.ds((c * NS + s) * (RPT // K), RPT // K)])


_deg_call = functools.partial(
    pl.kernel,
    out_type=jax.ShapeDtypeStruct((2 * P // K, K), jnp.float32),
    mesh=_MESH,
    compiler_params=pltpu.CompilerParams(use_tc_tiling_on_sc=False),
    scratch_types=[
        pltpu.VMEM((R16, K), jnp.int32),
        pltpu.VMEM((K,), jnp.int32),
        pltpu.VMEM((K, 16), jnp.float32),
        pltpu.VMEM((K, 16), jnp.float32),
        pltpu.VMEM((RPT, 16), jnp.float32),
        pltpu.VMEM((RPT // K, K), jnp.float32),
        pltpu.VMEM_SHARED((P, 16), jnp.float32),
        pltpu.SemaphoreType.DMA,
    ],
)(_sc_deg_body)


def _sc_agg128_body(hlo_hbm, hhi_hbm, e3_hbm, out_hbm,
                    srcall, dstall, xsrc, xdst,
                    rows0, rows1, rows2, rows3, acc,
                    gs0, gs1, gs2, gs3, ss0, ss1, ss2, ss3):
    c = lax.axis_index("c")
    s = lax.axis_index("s")

    pltpu.sync_copy(e3_hbm.at[0, pl.ds(s * R128, R128)], srcall)
    pltpu.sync_copy(e3_hbm.at[1, pl.ds(s * R128, R128)], dstall)

    @pl.when(c == 0)
    def _():
        pltpu.sync_copy(hlo_hbm.at[pl.ds(s * RPT, RPT)],
                        acc.at[pl.ds(s * RPT, RPT)])

    @pl.when(c == 1)
    def _():
        pltpu.sync_copy(hhi_hbm.at[pl.ds(s * RPT, RPT)],
                        acc.at[pl.ds(s * RPT, RPT)])

    plsc.subcore_barrier()

    def issue(b, buf, sem):
        @pl.when(c == 0)
        def _():
            pltpu.async_copy(hlo_hbm.at[srcall.at[b]], buf, sem)

        @pl.when(c == 1)
        def _():
            pltpu.async_copy(hhi_hbm.at[srcall.at[b]], buf, sem)

    _ring(issue, hlo_hbm, srcall, dstall, [rows0, rows1, rows2, rows3], acc,
          [gs0, gs1, gs2, gs3], [ss0, ss1, ss2, ss3], R128)

    @pl.when(s < 4)
    def _():
        pltpu.sync_copy(e3_hbm.at[0, ER - 4 + s], xsrc)
        pltpu.sync_copy(e3_hbm.at[1, ER - 4 + s], xdst)

        @pl.when(c == 0)
        def _():
            pltpu.async_copy(hlo_hbm.at[xsrc], rows0, gs0).wait()

        @pl.when(c == 1)
        def _():
            pltpu.async_copy(hhi_hbm.at[xsrc], rows0, gs0).wait()

        pltpu.sync_copy(rows0, acc.at[xdst], add=True)

    plsc.subcore_barrier()
    pltpu.sync_copy(acc.at[pl.ds(s * RPT, RPT)],
                    out_hbm.at[pl.ds(c * P + s * RPT, RPT)])


_agg128_call = functools.partial(
    pl.kernel,
    out_type=jax.ShapeDtypeStruct((2 * P, 64), jnp.bfloat16),
    mesh=_MESH,
    compiler_params=pltpu.CompilerParams(use_tc_tiling_on_sc=False),
    scratch_types=[
        pltpu.VMEM((R128, K), jnp.int32),
        pltpu.VMEM((R128, K), jnp.int32),
        pltpu.VMEM((K,), jnp.int32),
        pltpu.VMEM((K,), jnp.int32),
        pltpu.VMEM((K, 64), jnp.bfloat16),
        pltpu.VMEM((K, 64), jnp.bfloat16),
        pltpu.VMEM((K, 64), jnp.bfloat16),
        pltpu.VMEM((K, 64), jnp.bfloat16),
        pltpu.VMEM_SHARED((P, 64), jnp.bfloat16),
    ] + [pltpu.SemaphoreType.DMA] * 8,
)(_sc_agg128_body)


def _sc_agg16_body(h_hbm, e3_hbm, out_hbm,
                   srcall, dstall, xsrc, xdst,
                   rows0, rows1, rows2, rows3, acc,
                   gs0, gs1, gs2, gs3, ss0, ss1, ss2, ss3):
    c = lax.axis_index("c")
    s = lax.axis_index("s")
    wid = c * NS + s

    pltpu.sync_copy(e3_hbm.at[0, pl.ds(wid * R16, R16)], srcall)
    pltpu.sync_copy(e3_hbm.at[1, pl.ds(wid * R16, R16)], dstall)

    @pl.when(c == 0)
    def _():
        pltpu.sync_copy(h_hbm.at[pl.ds(s * RPT, RPT)],
                        acc.at[pl.ds(s * RPT, RPT)])

    @pl.when(c == 1)
    def _():
        _zero_rows_f32(rows0, K, 16)
        for t in range(RPT // K):
            pltpu.sync_copy(rows0, acc.at[pl.ds(s * RPT + t * K, K)])

    plsc.subcore_barrier()

    def issue(b, buf, sem):
        pltpu.async_copy(h_hbm.at[srcall.at[b]], buf, sem)

    RMAIN = 76
    _ring(issue, h_hbm, srcall, dstall, [rows0, rows1, rows2, rows3], acc,
          [gs0, gs1, gs2, gs3], [ss0, ss1, ss2, ss3], RMAIN)

    for b in (76, 77):
        pltpu.async_copy(h_hbm.at[srcall.at[b]], rows0, gs0).wait()
        pltpu.sync_copy(rows0, acc.at[dstall.at[b]], add=True)

    @pl.when(wid < 4)
    def _():
        pltpu.sync_copy(e3_hbm.at[0, ER - 4 + wid], xsrc)
        pltpu.sync_copy(e3_hbm.at[1, ER - 4 + wid], xdst)
        pltpu.async_copy(h_hbm.at[xsrc], rows0, gs0).wait()
        pltpu.sync_copy(rows0, acc.at[xdst], add=True)

    plsc.subcore_barrier()
    pltpu.sync_copy(acc.at[pl.ds(s * RPT, RPT)],
                    out_hbm.at[pl.ds(c * P + s * RPT, RPT)])


_agg16_call = functools.partial(
    pl.kernel,
    out_type=jax.ShapeDtypeStruct((2 * P, 16), jnp.float32),
    mesh=_MESH,
    compiler_params=pltpu.CompilerParams(use_tc_tiling_on_sc=False),
    scratch_types=[
        pltpu.VMEM((R16, K), jnp.int32),
        pltpu.VMEM((R16, K), jnp.int32),
        pltpu.VMEM((K,), jnp.int32),
        pltpu.VMEM((K,), jnp.int32),
        pltpu.VMEM((K, 16), jnp.float32),
        pltpu.VMEM((K, 16), jnp.float32),
        pltpu.VMEM((K, 16), jnp.float32),
        pltpu.VMEM((K, 16), jnp.float32),
        pltpu.VMEM_SHARED((P, 16), jnp.float32),
    ] + [pltpu.SemaphoreType.DMA] * 8,
)(_sc_agg16_body)


BLK = 1024
GRID = P // BLK
BLKO = 2048
GRIDO = P // BLKO


def _pk_to_col(d, nrows):
    g = nrows // K
    x = jnp.broadcast_to(d[:, None, :], (g, K, K)).reshape(nrows, K)
    l = lax.broadcasted_iota(jnp.int32, (nrows, K), 1)
    n = lax.broadcasted_iota(jnp.int32, (nrows, K), 0)
    return jnp.sum(jnp.where(l == n % K, x, 0.0), axis=1, keepdims=True)


def _tc_mm1a_body(x_ref, w1_ref, h_ref):
    i = pl.program_id(0)
    h = jnp.dot(x_ref[...], w1_ref[...], preferred_element_type=jnp.float32)
    row = i * BLK + lax.broadcasted_iota(jnp.int32, (BLK, 128), 0)
    h_ref[...] = jnp.where(row < N, h, 0.0)


def _tc_mm1a(x, W1):
    return pl.pallas_call(
        _tc_mm1a_body,
        grid=(GRID,),
        in_specs=[
            pl.BlockSpec((BLK, 128), lambda i: (i, 0)),
            pl.BlockSpec((128, 128), lambda i: (0, 0)),
        ],
        out_specs=pl.BlockSpec((BLK, 128), lambda i: (i, 0)),
        out_shape=jax.ShapeDtypeStruct((P, 128), jnp.float32),
    )(x, W1)


def _tc_mm1b_body(h_ref, d0_ref, d1_ref, hlo_ref, hhi_ref, dis_ref):
    dis = lax.rsqrt(d0_ref[...] + d1_ref[...] + 1.0)
    dis_ref[...] = dis
    h = h_ref[...] * _pk_to_col(dis, BLK)
    hb = h.astype(jnp.bfloat16)
    hlo_ref[...] = hb[:, :64]
    hhi_ref[...] = hb[:, 64:]


def _tc_mm1b(hraw, deg_pk):
    return pl.pallas_call(
        _tc_mm1b_body,
        grid=(GRID,),
        in_specs=[
            pl.BlockSpec((BLK, 128), lambda i: (i, 0)),
            pl.BlockSpec((BLK // K, K), lambda i: (i, 0)),
            pl.BlockSpec((BLK // K, K), lambda i: (i + GRID, 0)),
        ],
        out_specs=[
            pl.BlockSpec((BLK, 64), lambda i: (i, 0)),
            pl.BlockSpec((BLK, 64), lambda i: (i, 0)),
            pl.BlockSpec((BLK // K, K), lambda i: (i, 0)),
        ],
        out_shape=[
            jax.ShapeDtypeStruct((P, 64), jnp.bfloat16),
            jax.ShapeDtypeStruct((P, 64), jnp.bfloat16),
            jax.ShapeDtypeStruct((P // K, K), jnp.float32),
        ],
    )(hraw, deg_pk, deg_pk)


def _tc_mm2_body(alo_ref, ahi_ref, dis_ref, w2_ref, b1_ref, out_ref):
    dis1 = _pk_to_col(dis_ref[...], BLK)
    b1 = b1_ref[...]
    w2 = w2_ref[...]
    pre_lo = alo_ref[...].astype(jnp.float32) * dis1 + b1[:, :64]
    pre_hi = ahi_ref[...].astype(jnp.float32) * dis1 + b1[:, 64:]
    o_lo = jnp.maximum(pre_lo, 0.0)
    o_hi = jnp.maximum(pre_hi, 0.0)
    h2 = (jnp.dot(o_lo, w2[:64], preferred_element_type=jnp.float32)
          + jnp.dot(o_hi, w2[64:], preferred_element_type=jnp.float32))
    out_ref[...] = h2 * dis1


def _tc_mm2(a128, dis, W2, b1):
    return pl.pallas_call(
        _tc_mm2_body,
        grid=(GRID,),
        in_specs=[
            pl.BlockSpec((BLK, 64), lambda i: (i, 0)),
            pl.BlockSpec((BLK, 64), lambda i: (i + GRID, 0)),
            pl.BlockSpec((BLK // K, K), lambda i: (i, 0)),
            pl.BlockSpec((128, 16), lambda i: (0, 0)),
            pl.BlockSpec((1, 128), lambda i: (0, 0)),
        ],
        out_specs=pl.BlockSpec((BLK, 16), lambda i: (i, 0)),
        out_shape=jax.ShapeDtypeStruct((P, 16), jnp.float32),
    )(a128, a128, dis, W2, b1)


def _tc_out_body(p0_ref, p1_ref, dis_ref, b2_ref, out_ref):
    dis1 = _pk_to_col(dis_ref[...], BLKO)
    out_ref[...] = (p0_ref[...] + p1_ref[...]) * dis1 + b2_ref[...]


def _tc_out(acc2, dis, b2):
    return pl.pallas_call(
        _tc_out_body,
        grid=(GRIDO,),
        in_specs=[
            pl.BlockSpec((BLKO, 16), lambda i: (i, 0)),
            pl.BlockSpec((BLKO, 16), lambda i: (i + GRIDO, 0)),
            pl.BlockSpec((BLKO // K, K), lambda i: (i, 0)),
            pl.BlockSpec((1, 16), lambda i: (0, 0)),
        ],
        out_specs=pl.BlockSpec((BLKO, 16), lambda i: (i, 0)),
        out_shape=jax.ShapeDtypeStruct((N, 16), jnp.float32),
    )(acc2, acc2, dis, b2)


@jax.jit
def kernel(x, edge_index, W1, b1, W2, b2):
    e3 = edge_index.reshape(2, ER, K)

    deg_pk = _deg_call(e3)
    hraw = _tc_mm1a(x, W1)
    hlo, hhi, dis = _tc_mm1b(hraw, deg_pk)

    acc1 = _agg128_call(hlo, hhi, e3)
    h2p = _tc_mm2(acc1, dis, W2, b1.reshape(1, 128))

    acc2 = _agg16_call(h2p, e3)
    return _tc_out(acc2, dis, b2.reshape(1, 16))
